# baseline (speedup 1.0000x reference)
import jax
import jax.numpy as jnp
from jax.experimental import pallas as pl

BATCH = 4096
SEQLEN = 200
EMBED = 64


def _tc_body(x_ref, pos_ref, out_ref):
    x2 = x_ref[0]
    p = pos_ref[pl.program_id(0)]
    pt = jnp.broadcast_to(p[None, :], (128, 64)).T
    xe = x2[None, :, None, :]
    pe = pt.reshape(8, 8, 128)[:, None, :, :]
    out_ref[0] = xe + pe


def kernel(x, pos_table):
    xr = x.T.reshape(SEQLEN, 32, 128)
    out5 = pl.pallas_call(
        _tc_body,
        grid=(SEQLEN,),
        in_specs=[
            pl.BlockSpec((1, 32, 128), lambda l: (l, 0, 0)),
            pl.BlockSpec((SEQLEN, EMBED), lambda l: (0, 0)),
        ],
        out_specs=pl.BlockSpec((1, 8, 32, 8, 128), lambda l: (l, 0, 0, 0, 0)),
        out_shape=jax.ShapeDtypeStruct((SEQLEN, 8, 32, 8, 128), jnp.float32),
    )(xr, pos_table)
    t = out5.reshape(SEQLEN, 8, 32, 8, 128)
    return t.transpose(2, 4, 0, 1, 3).reshape(BATCH, SEQLEN, EMBED)


# speedup vs baseline: 2.0656x; 2.0656x over previous
import jax
import jax.numpy as jnp
from jax.experimental import pallas as pl

BATCH = 4096
SEQLEN = 200
EMBED = 64


BL = 8


def _tc_body(x_ref, pos_ref, out_ref):
    for i in range(BL):
        x2 = x_ref[i]
        p = pos_ref[pl.program_id(0) * BL + i]
        pt = jnp.broadcast_to(p[None, :], (128, 64)).T
        xe = x2[None, :, None, :]
        pe = pt.reshape(8, 8, 128)[:, None, :, :]
        out_ref[i] = xe + pe


def kernel(x, pos_table):
    xr = x.T.reshape(SEQLEN, 32, 128)
    out5 = pl.pallas_call(
        _tc_body,
        grid=(SEQLEN // BL,),
        in_specs=[
            pl.BlockSpec((BL, 32, 128), lambda l: (l, 0, 0)),
            pl.BlockSpec((SEQLEN, EMBED), lambda l: (0, 0)),
        ],
        out_specs=pl.BlockSpec((BL, 8, 32, 8, 128), lambda l: (l, 0, 0, 0, 0)),
        out_shape=jax.ShapeDtypeStruct((SEQLEN, 8, 32, 8, 128), jnp.float32),
    )(xr, pos_table)
    t = out5.reshape(SEQLEN, 8, 32, 8, 128)
    return t.transpose(2, 4, 0, 1, 3).reshape(BATCH, SEQLEN, EMBED)
